# Initial kernel scaffold; baseline (speedup 1.0000x reference)
import jax
import jax.numpy as jnp
from jax.experimental import pallas as pl


def kernel(source, weight):
    raise NotImplementedError("write your pallas kernel here")


# speedup vs baseline: 1.2847x; 1.2847x over previous
import functools

import jax
import jax.numpy as jnp
from jax import lax
from jax.experimental import pallas as pl
from jax.experimental.pallas import tpu as pltpu
from jax.experimental.pallas import tpu_sc as plsc


def _build_gather(B, D, n_ch, CH, num_cores, num_subcores):
    b_per_w = B // (num_cores * num_subcores)
    ch_per_w = b_per_w // CH
    mesh = plsc.VectorSubcoreMesh(core_axis_name="c", subcore_axis_name="s")

    @functools.partial(
        pl.kernel,
        mesh=mesh,
        out_type=jax.ShapeDtypeStruct((B, D), jnp.float32),
        scratch_types=[
            pltpu.VMEM((ch_per_w, CH), jnp.int32),
            pltpu.VMEM((CH, D), jnp.float32),
            pltpu.SemaphoreType.DMA,
        ],
    )
    def run(table_hbm, idx_hbm, out_hbm, idx_v, buf, sem):
        wid = lax.axis_index("s") * num_cores + lax.axis_index("c")
        base = wid * b_per_w
        pltpu.sync_copy(idx_hbm.at[pl.ds(wid * ch_per_w, ch_per_w)], idx_v)

        def body(c, carry):
            pltpu.async_copy(table_hbm.at[idx_v.at[c]], buf, sem).wait()
            pltpu.sync_copy(buf, out_hbm.at[pl.ds(base + c * CH, CH)])
            return carry

        lax.fori_loop(0, ch_per_w, body, 0)

    return run


def kernel(source, weight):
    SEQ, BATCH, NF = source.shape
    V, D = weight.shape
    B = SEQ * BATCH * NF
    idx = source.reshape(B).astype(jnp.int32)

    info = plsc.get_sparse_core_info()
    CH = 128
    idx2 = idx.reshape(B // CH, CH)

    run = _build_gather(B, D, B // CH, CH, info.num_cores, info.num_subcores)
    out = run(weight, idx2)
    return out.reshape(SEQ, BATCH, D)


# speedup vs baseline: 1.8889x; 1.4703x over previous
import functools

import jax
import jax.numpy as jnp
from jax import lax
from jax.experimental import pallas as pl
from jax.experimental.pallas import tpu as pltpu
from jax.experimental.pallas import tpu_sc as plsc


def _build_gather(B, D, n_ch, CH, num_cores, num_subcores):
    b_per_w = B // (num_cores * num_subcores)
    ch_per_w = b_per_w // CH
    mesh = plsc.VectorSubcoreMesh(core_axis_name="c", subcore_axis_name="s")

    n_pairs = b_per_w // CH // 2

    @functools.partial(
        pl.kernel,
        mesh=mesh,
        out_type=jax.ShapeDtypeStruct((B, D), jnp.float32),
        scratch_types=[
            pltpu.VMEM((ch_per_w, CH), jnp.int32),
            pltpu.VMEM((CH, D), jnp.float32),
            pltpu.VMEM((CH, D), jnp.float32),
            pltpu.SemaphoreType.DMA,
            pltpu.SemaphoreType.DMA,
        ],
    )
    def run(table_hbm, idx_hbm, out_hbm, idx_v, buf0, buf1, sem0, sem1):
        wid = lax.axis_index("s") * num_cores + lax.axis_index("c")
        base = wid * b_per_w
        pltpu.sync_copy(idx_hbm.at[pl.ds(wid * ch_per_w, ch_per_w)], idx_v)

        pltpu.async_copy(table_hbm.at[idx_v.at[0]], buf0, sem0)

        def body(j, carry):
            c = 2 * j
            h1 = pltpu.async_copy(table_hbm.at[idx_v.at[c + 1]], buf1, sem1)
            pltpu.make_async_copy(table_hbm.at[idx_v.at[c]], buf0, sem0).wait()
            pltpu.sync_copy(buf0, out_hbm.at[pl.ds(base + c * CH, CH)])

            @pl.when(j < n_pairs - 1)
            def _():
                pltpu.async_copy(table_hbm.at[idx_v.at[c + 2]], buf0, sem0)

            h1.wait()
            pltpu.sync_copy(buf1, out_hbm.at[pl.ds(base + (c + 1) * CH, CH)])
            return carry

        lax.fori_loop(0, n_pairs, body, 0)

    return run


def kernel(source, weight):
    SEQ, BATCH, NF = source.shape
    V, D = weight.shape
    B = SEQ * BATCH * NF
    idx = source.reshape(B).astype(jnp.int32)

    info = plsc.get_sparse_core_info()
    CH = 128
    idx2 = idx.reshape(B // CH, CH)

    run = _build_gather(B, D, B // CH, CH, info.num_cores, info.num_subcores)
    out = run(weight, idx2)
    return out.reshape(SEQ, BATCH, D)
